# baseline (speedup 1.0000x reference)
import functools

import jax
import jax.numpy as jnp
from jax import lax
from jax.experimental import pallas as pl
from jax.experimental.pallas import tpu as pltpu, tpu_sc as plsc

VOCAB = 1000000
EMB = 64
N_PART = 4
PART = EMB // N_PART
BATCH = 16384
HIST = 50
B = BATCH * HIST

NW = 32
B_PER_W = B // NW
CHUNK = 1600
N_CHUNKS = B_PER_W // CHUNK

TR_N = 8192
TR_STEPS = (VOCAB + TR_N - 1) // TR_N


def _tr_body(in_ref, out_ref):
    x = in_ref[...]
    t = jnp.transpose(x, (1, 0))
    t3 = t.reshape(TR_N // 8, 8, PART)
    for r in range(8):
        out_ref[:, PART * r : PART * (r + 1)] = t3[:, r, :]


def _relayout(wt):
    q = pl.pallas_call(
        _tr_body,
        grid=(TR_STEPS,),
        in_specs=[pl.BlockSpec((16, TR_N), lambda i: (0, i))],
        out_specs=pl.BlockSpec((TR_N // 8, 128), lambda i: (i, 0)),
        out_shape=jax.ShapeDtypeStruct((VOCAB // 8, 128), jnp.float32),
    )(wt)
    return q.reshape(VOCAB, PART)


def _make_gather():
    mesh = plsc.VectorSubcoreMesh(core_axis_name="c", subcore_axis_name="s")

    @functools.partial(
        pl.kernel,
        mesh=mesh,
        out_type=jax.ShapeDtypeStruct((B, PART), jnp.float32),
        scratch_types=[
            pltpu.VMEM((CHUNK,), jnp.int32),
            pltpu.VMEM((CHUNK, PART), jnp.float32),
            pltpu.VMEM((CHUNK, PART), jnp.float32),
            pltpu.SemaphoreType.DMA,
            pltpu.SemaphoreType.DMA,
        ],
        compiler_params=pltpu.CompilerParams(use_tc_tiling_on_sc=False),
    )
    def emb_kernel(idx_hbm, w, out_hbm, idx_v, r0, r1, sem0, sem1):
        wid = lax.axis_index("s") * 2 + lax.axis_index("c")
        base = wid * B_PER_W

        def fetch(ci, buf, sem):
            row0 = base + ci * CHUNK
            pltpu.sync_copy(idx_hbm.at[pl.ds(row0, CHUNK)], idx_v)
            return pltpu.async_copy(w.at[idx_v], buf, sem)

        def body(ci, _):
            row0 = base + ci * CHUNK

            @pl.when(ci % 2 == 0)
            def _():
                pltpu.make_async_copy(w.at[idx_v], r0, sem0).wait()
                fetch_next(ci, r1, sem1)
                pltpu.sync_copy(r0, out_hbm.at[pl.ds(row0, CHUNK)])

            @pl.when(ci % 2 == 1)
            def _():
                pltpu.make_async_copy(w.at[idx_v], r1, sem1).wait()
                fetch_next(ci, r0, sem0)
                pltpu.sync_copy(r1, out_hbm.at[pl.ds(row0, CHUNK)])

            return ()

        def fetch_next(ci, buf, sem):
            @pl.when(ci + 1 < N_CHUNKS)
            def _():
                fetch(ci + 1, buf, sem)

        fetch(0, r0, sem0)
        lax.fori_loop(0, N_CHUNKS, body, ())

    return emb_kernel


_gather = _make_gather()


def kernel(x, W0, W1, W2, W3):
    idx = x.reshape(-1).astype(jnp.int32)
    outs = []
    for W in (W0, W1, W2, W3):
        q = _relayout(W.T)
        outs.append(_gather(idx, q))
    out = jnp.concatenate(outs, axis=-1)
    return out.reshape(BATCH, HIST, EMB)


# speedup vs baseline: 1.8415x; 1.8415x over previous
import functools

import jax
import jax.numpy as jnp
from jax import lax
from jax.experimental import pallas as pl
from jax.experimental.pallas import tpu as pltpu, tpu_sc as plsc

VOCAB = 1000000
EMB = 64
N_PART = 4
PART = EMB // N_PART
BATCH = 16384
HIST = 50
B = BATCH * HIST

NW = 32
B_PER_W = B // NW
CHUNK = 800
N_CHUNKS = B_PER_W // CHUNK

TR_N = 8192
TR_STEPS = (VOCAB + TR_N - 1) // TR_N


def _tr_body(in_ref, out_ref):
    x = in_ref[...]
    t = jnp.transpose(x, (1, 0))
    t3 = t.reshape(TR_N // 8, 8, PART)
    for r in range(8):
        out_ref[:, PART * r : PART * (r + 1)] = t3[:, r, :]


def _relayout(wt):
    q = pl.pallas_call(
        _tr_body,
        grid=(TR_STEPS,),
        in_specs=[pl.BlockSpec((16, TR_N), lambda i: (0, i))],
        out_specs=pl.BlockSpec((TR_N // 8, 128), lambda i: (i, 0)),
        out_shape=jax.ShapeDtypeStruct((VOCAB // 8, 128), jnp.float32),
    )(wt)
    return q.reshape(VOCAB, PART)


def _make_gather():
    mesh = plsc.VectorSubcoreMesh(core_axis_name="c", subcore_axis_name="s")

    @functools.partial(
        pl.kernel,
        mesh=mesh,
        out_type=jax.ShapeDtypeStruct((B, EMB), jnp.float32),
        scratch_types=[
            pltpu.VMEM((CHUNK,), jnp.int32),
            pltpu.VMEM((CHUNK,), jnp.int32),
            [pltpu.VMEM((CHUNK, PART), jnp.float32) for _ in range(4)],
            [pltpu.VMEM((CHUNK, PART), jnp.float32) for _ in range(4)],
            pltpu.SemaphoreType.DMA,
            pltpu.SemaphoreType.DMA,
        ],
        compiler_params=pltpu.CompilerParams(use_tc_tiling_on_sc=False),
    )
    def emb_kernel(idx_hbm, w0, w1, w2, w3, out_hbm,
                   idx_a, idx_b, bufs_a, bufs_b, sem_a, sem_b):
        wid = lax.axis_index("s") * 2 + lax.axis_index("c")
        base = wid * B_PER_W
        ws = (w0, w1, w2, w3)

        def fetch(ci, idx_v, bufs, sem):
            row0 = base + ci * CHUNK
            pltpu.sync_copy(idx_hbm.at[pl.ds(row0, CHUNK)], idx_v)
            for t in range(4):
                pltpu.async_copy(ws[t].at[idx_v], bufs[t], sem)

        def drain_and_write(ci, idx_v, bufs, sem):
            row0 = base + ci * CHUNK
            for t in range(4):
                pltpu.make_async_copy(ws[t].at[idx_v], bufs[t], sem).wait()
            for t in range(4):
                pltpu.sync_copy(
                    bufs[t],
                    out_hbm.at[pl.ds(row0, CHUNK), pl.ds(t * PART, PART)],
                )

        def body(ci, _):
            @pl.when(ci % 2 == 0)
            def _():
                fetch_next(ci, idx_b, bufs_b, sem_b)
                drain_and_write(ci, idx_a, bufs_a, sem_a)

            @pl.when(ci % 2 == 1)
            def _():
                fetch_next(ci, idx_a, bufs_a, sem_a)
                drain_and_write(ci, idx_b, bufs_b, sem_b)

            return ()

        def fetch_next(ci, idx_v, bufs, sem):
            @pl.when(ci + 1 < N_CHUNKS)
            def _():
                fetch(ci + 1, idx_v, bufs, sem)

        fetch(0, idx_a, bufs_a, sem_a)
        lax.fori_loop(0, N_CHUNKS, body, ())

    return emb_kernel


_gather = _make_gather()


def kernel(x, W0, W1, W2, W3):
    idx = x.reshape(-1).astype(jnp.int32)
    q0 = _relayout(W0.T)
    q1 = _relayout(W1.T)
    q2 = _relayout(W2.T)
    q3 = _relayout(W3.T)
    out = _gather(idx, q0, q1, q2, q3)
    return out.reshape(BATCH, HIST, EMB)
